# baseline (speedup 1.0000x reference)
import functools
import math

import jax
import jax.numpy as jnp
from jax import lax
from jax.experimental import pallas as pl
from jax.experimental.pallas import tpu as pltpu
from jax.experimental.pallas import tpu_sc as plsc

_N_NODES = 10000
_N_EDGES = 320000
_D = 128

_NC = 2
_NS = 16
_NW = _NC * _NS
_EPW = _N_EDGES // _NW
_K = 80
_NCHUNK = _EPW // _K
_N_PAD = 10240
_ROWS_PER_TILE = _N_PAD // _NS


def _radial_body(eet_ref, eft_ref, w1_ref, b1_ref, w2_ref, b2_ref, w3_ref,
                 b3_ref, o_ref):
    ee = jnp.transpose(eet_ref[...], (1, 0))
    ef = jnp.transpose(eft_ref[...], (1, 0))
    h = jnp.dot(ee, w1_ref[...], preferred_element_type=jnp.float32)
    h = h + b1_ref[...]
    h = h * jax.nn.sigmoid(h)
    h = jnp.dot(h, w2_ref[...], preferred_element_type=jnp.float32)
    h = h + b2_ref[...]
    h = h * jax.nn.sigmoid(h)
    w = jnp.dot(h, w3_ref[...], preferred_element_type=jnp.float32)
    w = w + b3_ref[...]
    o_ref[...] = w * ef


def _radial_weights(edge_embed_t, edge_feat_t, W1, b1, W2, b2, W3, b3):
    bm = 2560
    grid = _N_EDGES // bm
    full = lambda s: pl.BlockSpec(s, lambda i: (0, 0))
    return pl.pallas_call(
        _radial_body,
        grid=(grid,),
        in_specs=[
            pl.BlockSpec((16, bm), lambda i: (0, i)),
            pl.BlockSpec((1, bm), lambda i: (0, i)),
            full((16, 64)), full((1, 64)),
            full((64, 16)), full((1, 16)),
            full((16, 128)), full((1, 128)),
        ],
        out_specs=pl.BlockSpec((bm, _D), lambda i: (i, 0)),
        out_shape=jax.ShapeDtypeStruct((_N_EDGES, _D), jnp.float32),
    )(edge_embed_t, edge_feat_t, W1, b1.reshape(1, -1), W2, b2.reshape(1, -1),
      W3, b3.reshape(1, -1))


def _sc_body(src_hbm, dst_hbm, nf_hbm, w_hbm, zero_hbm, out0_hbm, out1_hbm,
             srcv0, dstv0, srcv1, dstv1, srcv2, dstv2,
             rows0, wv0, rows1, wv1,
             acc, si0, si1, si2, sd0, sd1, ss0, ss1):
    cid = lax.axis_index("c")
    sid = lax.axis_index("s")
    wid = sid * _NC + cid

    idxb = ((srcv0, dstv0, si0), (srcv1, dstv1, si1), (srcv2, dstv2, si2))
    datb = ((rows0, wv0, sd0, ss0), (rows1, wv1, sd1, ss1))

    def start_idx(c, ib):
        srcv, dstv, si = idxb[ib]
        base = wid * _EPW + c * _K
        pltpu.async_copy(src_hbm.at[pl.ds(base, _K)], srcv, si)
        pltpu.async_copy(dst_hbm.at[pl.ds(base, _K)], dstv, si)

    def wait_idx(ib):
        srcv, dstv, si = idxb[ib]
        pltpu.make_async_copy(src_hbm.at[pl.ds(0, _K)], srcv, si).wait()
        pltpu.make_async_copy(dst_hbm.at[pl.ds(0, _K)], dstv, si).wait()

    def start_dat(c, db, ib):
        srcv = idxb[ib][0]
        rows, wv, sd, _ = datb[db]
        base = wid * _EPW + c * _K
        pltpu.async_copy(nf_hbm.at[srcv], rows, sd)
        pltpu.async_copy(w_hbm.at[pl.ds(base, _K)], wv, sd)

    def wait_dat(db, ib):
        srcv = idxb[ib][0]
        rows, wv, sd, _ = datb[db]
        pltpu.make_async_copy(nf_hbm.at[srcv], rows, sd).wait()
        pltpu.make_async_copy(w_hbm.at[pl.ds(0, _K)], wv, sd).wait()

    def compute(db):
        rows, wv, _, _ = datb[db]

        def mul(j, carry2):
            for l in range(_D // 16):
                s = pl.ds(l * 16, 16)
                rows[j, s] = rows[j, s] * wv[j, s]
            return carry2

        lax.fori_loop(0, _K, mul, 0, unroll=8)

    def start_sct(db, ib):
        dstv = idxb[ib][1]
        rows, _, _, ss = datb[db]
        pltpu.async_copy(rows, acc.at[dstv], ss, add=True)

    def wait_sct(db, ib):
        dstv = idxb[ib][1]
        rows, _, _, ss = datb[db]
        pltpu.make_async_copy(rows, acc.at[dstv], ss).wait()

    stripe = pl.ds(sid * _ROWS_PER_TILE, _ROWS_PER_TILE)
    pltpu.sync_copy(zero_hbm.at[stripe], acc.at[stripe])
    plsc.subcore_barrier()

    start_idx(0, 0)
    start_idx(1, 1)
    wait_idx(0)
    start_dat(0, 0, 0)

    def step(c, k):
        ib, db = k % 3, k % 2
        ib1, db1 = (k + 1) % 3, (k + 1) % 2
        ib2 = (k + 2) % 3

        @pl.when(c < _NCHUNK)
        def _():
            wait_dat(db, ib)

            @pl.when(c + 1 < _NCHUNK)
            def _():
                wait_idx(ib1)

                @pl.when(c > 0)
                def _():
                    wait_sct(db1, (k + 2) % 3)

                start_dat(c + 1, db1, ib1)

            compute(db)
            start_sct(db, ib)

            @pl.when(c + 2 < _NCHUNK)
            def _():
                start_idx(c + 2, ib2)

    def six(p, carry):
        for k in range(6):
            step(6 * p + k, k)
        return carry

    lax.fori_loop(0, (_NCHUNK + 5) // 6, six, 0)

    wait_sct((_NCHUNK - 1) % 2, (_NCHUNK - 1) % 3)
    if _NCHUNK > 1:
        wait_sct((_NCHUNK - 2) % 2, (_NCHUNK - 2) % 3)

    plsc.subcore_barrier()

    @pl.when(cid == 0)
    def _():
        pltpu.sync_copy(acc.at[stripe], out0_hbm.at[stripe])

    @pl.when(cid == 1)
    def _():
        pltpu.sync_copy(acc.at[stripe], out1_hbm.at[stripe])


def _sc_scatter(src, dst, node_feat, w_eff, zeros):
    mesh = plsc.VectorSubcoreMesh(core_axis_name="c", subcore_axis_name="s")
    f = pl.kernel(
        _sc_body,
        out_type=(jax.ShapeDtypeStruct((_N_PAD, _D), jnp.float32),
                  jax.ShapeDtypeStruct((_N_PAD, _D), jnp.float32)),
        mesh=mesh,
        scratch_types=[
            pltpu.VMEM((_K,), jnp.int32),
            pltpu.VMEM((_K,), jnp.int32),
            pltpu.VMEM((_K,), jnp.int32),
            pltpu.VMEM((_K,), jnp.int32),
            pltpu.VMEM((_K,), jnp.int32),
            pltpu.VMEM((_K,), jnp.int32),
            pltpu.VMEM((_K, _D), jnp.float32),
            pltpu.VMEM((_K, _D), jnp.float32),
            pltpu.VMEM((_K, _D), jnp.float32),
            pltpu.VMEM((_K, _D), jnp.float32),
            pltpu.VMEM_SHARED((_N_PAD, _D), jnp.float32),
        ] + [pltpu.SemaphoreType.DMA] * 7,
    )
    return f(src, dst, node_feat, w_eff, zeros)


def _combine_body(p0_ref, p1_ref, nf_ref, wsc_ref, o_ref):
    sc = jnp.dot(nf_ref[...], wsc_ref[...], preferred_element_type=jnp.float32)
    o_ref[...] = p0_ref[...] + p1_ref[...] + sc * (1.0 / math.sqrt(_D))


def _combine(p0, p1, node_feat, Wsc):
    bn = 2000
    grid = _N_NODES // bn
    return pl.pallas_call(
        _combine_body,
        grid=(grid,),
        in_specs=[
            pl.BlockSpec((bn, _D), lambda i: (i, 0)),
            pl.BlockSpec((bn, _D), lambda i: (i, 0)),
            pl.BlockSpec((bn, _D), lambda i: (i, 0)),
            pl.BlockSpec((_D, _D), lambda i: (0, 0)),
        ],
        out_specs=pl.BlockSpec((bn, _D), lambda i: (i, 0)),
        out_shape=jax.ShapeDtypeStruct((_N_NODES, _D), jnp.float32),
    )(p0, p1, node_feat, Wsc)


def kernel(edge_index, node_feat, edge_feat, edge_embed, dim_size,
           W1, b1, W2, b2, W3, b3, Wsc):
    del dim_size
    src = edge_index[0]
    dst = edge_index[1]
    w_eff = _radial_weights(edge_embed.T, edge_feat.T, W1, b1, W2, b2, W3, b3)
    zeros = jnp.zeros((_N_PAD, _D), jnp.float32)
    p0, p1 = _sc_scatter(src, dst, node_feat, w_eff, zeros)
    return _combine(p0, p1, node_feat, Wsc)


# speedup vs baseline: 1.0079x; 1.0079x over previous
import functools
import math

import jax
import jax.numpy as jnp
from jax import lax
from jax.experimental import pallas as pl
from jax.experimental.pallas import tpu as pltpu
from jax.experimental.pallas import tpu_sc as plsc

_N_NODES = 10000
_N_EDGES = 320000
_D = 128

_NC = 2
_NS = 16
_NW = _NC * _NS
_EPW = _N_EDGES // _NW
_K = 80
_NCHUNK = _EPW // _K
_N_PAD = 10240
_ROWS_PER_TILE = _N_PAD // _NS


def _radial_body(eet_ref, eft_ref, w1_ref, b1_ref, w2_ref, b2_ref, w3_ref,
                 b3_ref, o_ref):
    ee = jnp.transpose(eet_ref[...], (1, 0))
    ef = jnp.transpose(eft_ref[...], (1, 0))
    h = jnp.dot(ee, w1_ref[...], preferred_element_type=jnp.float32)
    h = h + b1_ref[...]
    h = h * jax.nn.sigmoid(h)
    h = jnp.dot(h, w2_ref[...], preferred_element_type=jnp.float32)
    h = h + b2_ref[...]
    h = h * jax.nn.sigmoid(h)
    w = jnp.dot(h, w3_ref[...], preferred_element_type=jnp.float32)
    w = w + b3_ref[...]
    o_ref[...] = w * ef


def _radial_weights(edge_embed_t, edge_feat_t, W1, b1, W2, b2, W3, b3):
    bm = 2560
    grid = _N_EDGES // bm
    full = lambda s: pl.BlockSpec(s, lambda i: (0, 0))
    return pl.pallas_call(
        _radial_body,
        grid=(grid,),
        in_specs=[
            pl.BlockSpec((16, bm), lambda i: (0, i)),
            pl.BlockSpec((1, bm), lambda i: (0, i)),
            full((16, 64)), full((1, 64)),
            full((64, 16)), full((1, 16)),
            full((16, 128)), full((1, 128)),
        ],
        out_specs=pl.BlockSpec((bm, _D), lambda i: (i, 0)),
        out_shape=jax.ShapeDtypeStruct((_N_EDGES, _D), jnp.float32),
    )(edge_embed_t, edge_feat_t, W1, b1.reshape(1, -1), W2, b2.reshape(1, -1),
      W3, b3.reshape(1, -1))


def _sc_body(src_hbm, dst_hbm, nf_hbm, w_hbm, zero_hbm, out0_hbm, out1_hbm,
             srcv0, dstv0, srcv1, dstv1, srcv2, dstv2,
             rows0, wv0, rows1, wv1,
             acc, si0, si1, si2, sd0, sd1, ss0, ss1):
    cid = lax.axis_index("c")
    sid = lax.axis_index("s")
    wid = sid * _NC + cid

    idxb = ((srcv0, dstv0, si0), (srcv1, dstv1, si1), (srcv2, dstv2, si2))
    datb = ((rows0, wv0, sd0, ss0), (rows1, wv1, sd1, ss1))

    def start_idx(c, ib):
        srcv, dstv, si = idxb[ib]
        base = wid * _EPW + c * _K
        pltpu.async_copy(src_hbm.at[pl.ds(base, _K)], srcv, si)
        pltpu.async_copy(dst_hbm.at[pl.ds(base, _K)], dstv, si)

    def wait_idx(ib):
        srcv, dstv, si = idxb[ib]
        pltpu.make_async_copy(src_hbm.at[pl.ds(0, _K)], srcv, si).wait()
        pltpu.make_async_copy(dst_hbm.at[pl.ds(0, _K)], dstv, si).wait()

    def start_dat(c, db, ib):
        srcv = idxb[ib][0]
        rows, wv, sd, _ = datb[db]
        base = wid * _EPW + c * _K
        pltpu.async_copy(nf_hbm.at[srcv], rows, sd)
        pltpu.async_copy(w_hbm.at[pl.ds(base, _K)], wv, sd)

    def wait_dat(db, ib):
        srcv = idxb[ib][0]
        rows, wv, sd, _ = datb[db]
        pltpu.make_async_copy(nf_hbm.at[srcv], rows, sd).wait()
        pltpu.make_async_copy(w_hbm.at[pl.ds(0, _K)], wv, sd).wait()

    def compute(db):
        rows, wv, _, _ = datb[db]

        def mul(j, carry2):
            for l in range(_D // 16):
                s = pl.ds(l * 16, 16)
                rows[j, s] = rows[j, s] * wv[j, s]
            return carry2

        lax.fori_loop(0, _K, mul, 0, unroll=4)

    def start_sct(db, ib):
        dstv = idxb[ib][1]
        rows, _, _, ss = datb[db]
        pltpu.async_copy(rows, acc.at[dstv], ss, add=True)

    def wait_sct(db, ib):
        dstv = idxb[ib][1]
        rows, _, _, ss = datb[db]
        pltpu.make_async_copy(rows, acc.at[dstv], ss).wait()

    stripe = pl.ds(sid * _ROWS_PER_TILE, _ROWS_PER_TILE)
    pltpu.sync_copy(zero_hbm.at[stripe], acc.at[stripe])
    plsc.subcore_barrier()

    start_idx(0, 0)
    start_idx(1, 1)
    wait_idx(0)
    start_dat(0, 0, 0)

    def step(c, k):
        ib, db = k % 3, k % 2
        ib1, db1 = (k + 1) % 3, (k + 1) % 2
        ib2 = (k + 2) % 3

        @pl.when(c < _NCHUNK)
        def _():
            wait_dat(db, ib)

            @pl.when(c + 1 < _NCHUNK)
            def _():
                wait_idx(ib1)

                @pl.when(c > 0)
                def _():
                    wait_sct(db1, (k + 2) % 3)

                start_dat(c + 1, db1, ib1)

            compute(db)
            start_sct(db, ib)

            @pl.when(c + 2 < _NCHUNK)
            def _():
                start_idx(c + 2, ib2)

    def six(p, carry):
        for k in range(6):
            step(6 * p + k, k)
        return carry

    lax.fori_loop(0, (_NCHUNK + 5) // 6, six, 0)

    wait_sct((_NCHUNK - 1) % 2, (_NCHUNK - 1) % 3)
    if _NCHUNK > 1:
        wait_sct((_NCHUNK - 2) % 2, (_NCHUNK - 2) % 3)

    plsc.subcore_barrier()

    @pl.when(cid == 0)
    def _():
        pltpu.sync_copy(acc.at[stripe], out0_hbm.at[stripe])

    @pl.when(cid == 1)
    def _():
        pltpu.sync_copy(acc.at[stripe], out1_hbm.at[stripe])


def _sc_scatter(src, dst, node_feat, w_eff, zeros):
    mesh = plsc.VectorSubcoreMesh(core_axis_name="c", subcore_axis_name="s")
    f = pl.kernel(
        _sc_body,
        out_type=(jax.ShapeDtypeStruct((_N_PAD, _D), jnp.float32),
                  jax.ShapeDtypeStruct((_N_PAD, _D), jnp.float32)),
        mesh=mesh,
        scratch_types=[
            pltpu.VMEM((_K,), jnp.int32),
            pltpu.VMEM((_K,), jnp.int32),
            pltpu.VMEM((_K,), jnp.int32),
            pltpu.VMEM((_K,), jnp.int32),
            pltpu.VMEM((_K,), jnp.int32),
            pltpu.VMEM((_K,), jnp.int32),
            pltpu.VMEM((_K, _D), jnp.float32),
            pltpu.VMEM((_K, _D), jnp.float32),
            pltpu.VMEM((_K, _D), jnp.float32),
            pltpu.VMEM((_K, _D), jnp.float32),
            pltpu.VMEM_SHARED((_N_PAD, _D), jnp.float32),
        ] + [pltpu.SemaphoreType.DMA] * 7,
    )
    return f(src, dst, node_feat, w_eff, zeros)


def _combine_body(p0_ref, p1_ref, nf_ref, wsc_ref, o_ref):
    sc = jnp.dot(nf_ref[...], wsc_ref[...], preferred_element_type=jnp.float32)
    o_ref[...] = p0_ref[...] + p1_ref[...] + sc * (1.0 / math.sqrt(_D))


def _combine(p0, p1, node_feat, Wsc):
    bn = 2000
    grid = _N_NODES // bn
    return pl.pallas_call(
        _combine_body,
        grid=(grid,),
        in_specs=[
            pl.BlockSpec((bn, _D), lambda i: (i, 0)),
            pl.BlockSpec((bn, _D), lambda i: (i, 0)),
            pl.BlockSpec((bn, _D), lambda i: (i, 0)),
            pl.BlockSpec((_D, _D), lambda i: (0, 0)),
        ],
        out_specs=pl.BlockSpec((bn, _D), lambda i: (i, 0)),
        out_shape=jax.ShapeDtypeStruct((_N_NODES, _D), jnp.float32),
    )(p0, p1, node_feat, Wsc)


def kernel(edge_index, node_feat, edge_feat, edge_embed, dim_size,
           W1, b1, W2, b2, W3, b3, Wsc):
    del dim_size
    src = edge_index[0]
    dst = edge_index[1]
    w_eff = _radial_weights(edge_embed.T, edge_feat.T, W1, b1, W2, b2, W3, b3)
    zeros = jnp.zeros((_N_PAD, _D), jnp.float32)
    p0, p1 = _sc_scatter(src, dst, node_feat, w_eff, zeros)
    return _combine(p0, p1, node_feat, Wsc)


# speedup vs baseline: 1.0913x; 1.0827x over previous
import math

import jax
import jax.numpy as jnp
from jax import lax
from jax.experimental import pallas as pl
from jax.experimental.pallas import tpu as pltpu
from jax.experimental.pallas import tpu_sc as plsc

_N_NODES = 10000
_N_EDGES = 320000
_D = 128

_NC = 2
_NS = 16
_NW = _NC * _NS
_EPW = _N_EDGES // _NW
_K = 80
_NCHUNK = _EPW // _K
_N_PAD = 10240
_ROWS_PER_TILE = _N_PAD // _NS


def _radial_body(eet_ref, eft_ref, w1_ref, b1_ref, w2_ref, b2_ref, w3_ref,
                 b3_ref, o_ref):
    ee = jnp.transpose(eet_ref[...], (1, 0))
    ef = jnp.transpose(eft_ref[...], (1, 0))
    h = jnp.dot(ee, w1_ref[...], preferred_element_type=jnp.float32)
    h = h + b1_ref[...]
    h = h * jax.nn.sigmoid(h)
    h = jnp.dot(h, w2_ref[...], preferred_element_type=jnp.float32)
    h = h + b2_ref[...]
    h = h * jax.nn.sigmoid(h)
    w = jnp.dot(h, w3_ref[...], preferred_element_type=jnp.float32)
    w = w + b3_ref[...]
    o_ref[...] = w * ef


_BM = 2560


def _radial_weights(edge_embed_t, edge_feat_t, W1, b1, W2, b2, W3, b3,
                    n_rows, blk0):
    grid = n_rows // _BM
    full = lambda s: pl.BlockSpec(s, lambda i: (0, 0))
    return pl.pallas_call(
        _radial_body,
        grid=(grid,),
        in_specs=[
            pl.BlockSpec((16, _BM), lambda i: (0, i + blk0)),
            pl.BlockSpec((1, _BM), lambda i: (0, i + blk0)),
            full((16, 64)), full((1, 64)),
            full((64, 16)), full((1, 16)),
            full((16, 128)), full((1, 128)),
        ],
        out_specs=pl.BlockSpec((_BM, _D), lambda i: (i, 0)),
        out_shape=jax.ShapeDtypeStruct((n_rows, _D), jnp.float32),
    )(edge_embed_t, edge_feat_t, W1, b1.reshape(1, -1), W2, b2.reshape(1, -1),
      W3, b3.reshape(1, -1))


def _sc_body(epw, nchunk, edge0,
             src_hbm, dst_hbm, nf_hbm, w_hbm, zero_hbm, out0_hbm, out1_hbm,
             srcv0, dstv0, srcv1, dstv1, srcv2, dstv2,
             rows0, wv0, rows1, wv1,
             acc, si0, si1, si2, sd0, sd1, ss0, ss1):
    cid = lax.axis_index("c")
    sid = lax.axis_index("s")
    wid = sid * _NC + cid

    idxb = ((srcv0, dstv0, si0), (srcv1, dstv1, si1), (srcv2, dstv2, si2))
    datb = ((rows0, wv0, sd0, ss0), (rows1, wv1, sd1, ss1))

    def start_idx(c, ib):
        srcv, dstv, si = idxb[ib]
        base = edge0 + wid * epw + c * _K
        pltpu.async_copy(src_hbm.at[pl.ds(base, _K)], srcv, si)
        pltpu.async_copy(dst_hbm.at[pl.ds(base, _K)], dstv, si)

    def wait_idx(ib):
        srcv, dstv, si = idxb[ib]
        pltpu.make_async_copy(src_hbm.at[pl.ds(0, _K)], srcv, si).wait()
        pltpu.make_async_copy(dst_hbm.at[pl.ds(0, _K)], dstv, si).wait()

    def start_dat(c, db, ib):
        srcv = idxb[ib][0]
        rows, wv, sd, _ = datb[db]
        base = wid * epw + c * _K
        pltpu.async_copy(nf_hbm.at[srcv], rows, sd)
        pltpu.async_copy(w_hbm.at[pl.ds(base, _K)], wv, sd)

    def wait_dat(db, ib):
        srcv = idxb[ib][0]
        rows, wv, sd, _ = datb[db]
        pltpu.make_async_copy(nf_hbm.at[srcv], rows, sd).wait()
        pltpu.make_async_copy(w_hbm.at[pl.ds(0, _K)], wv, sd).wait()

    def compute(db):
        rows, wv, _, _ = datb[db]

        def mul(j, carry2):
            for l in range(_D // 16):
                s = pl.ds(l * 16, 16)
                rows[j, s] = rows[j, s] * wv[j, s]
            return carry2

        lax.fori_loop(0, _K, mul, 0, unroll=4)

    def start_sct(db, ib):
        dstv = idxb[ib][1]
        rows, _, _, ss = datb[db]
        pltpu.async_copy(rows, acc.at[dstv], ss, add=True)

    def wait_sct(db, ib):
        dstv = idxb[ib][1]
        rows, _, _, ss = datb[db]
        pltpu.make_async_copy(rows, acc.at[dstv], ss).wait()

    stripe = pl.ds(sid * _ROWS_PER_TILE, _ROWS_PER_TILE)
    pltpu.sync_copy(zero_hbm.at[stripe], acc.at[stripe])
    plsc.subcore_barrier()

    start_idx(0, 0)
    start_idx(1, 1)
    wait_idx(0)
    start_dat(0, 0, 0)

    def step(c, k):
        ib, db = k % 3, k % 2
        ib1, db1 = (k + 1) % 3, (k + 1) % 2
        ib2 = (k + 2) % 3

        @pl.when(c < nchunk)
        def _():
            wait_dat(db, ib)

            @pl.when(c + 1 < nchunk)
            def _():
                wait_idx(ib1)

                @pl.when(c > 0)
                def _():
                    wait_sct(db1, (k + 2) % 3)

                start_dat(c + 1, db1, ib1)

            compute(db)
            start_sct(db, ib)

            @pl.when(c + 2 < nchunk)
            def _():
                start_idx(c + 2, ib2)

    def six(p, carry):
        for k in range(6):
            step(6 * p + k, k)
        return carry

    lax.fori_loop(0, (nchunk + 5) // 6, six, 0)

    wait_sct((nchunk - 1) % 2, (nchunk - 1) % 3)
    if nchunk > 1:
        wait_sct((nchunk - 2) % 2, (nchunk - 2) % 3)

    plsc.subcore_barrier()

    @pl.when(cid == 0)
    def _():
        pltpu.sync_copy(acc.at[stripe], out0_hbm.at[stripe])

    @pl.when(cid == 1)
    def _():
        pltpu.sync_copy(acc.at[stripe], out1_hbm.at[stripe])


def _sc_scatter(src, dst, node_feat, w_eff, zeros, epw, nchunk, edge0):
    import functools as _ft
    mesh = plsc.VectorSubcoreMesh(core_axis_name="c", subcore_axis_name="s")
    f = pl.kernel(
        _ft.partial(_sc_body, epw, nchunk, edge0),
        out_type=(jax.ShapeDtypeStruct((_N_PAD, _D), jnp.float32),
                  jax.ShapeDtypeStruct((_N_PAD, _D), jnp.float32)),
        mesh=mesh,
        scratch_types=[
            pltpu.VMEM((_K,), jnp.int32),
            pltpu.VMEM((_K,), jnp.int32),
            pltpu.VMEM((_K,), jnp.int32),
            pltpu.VMEM((_K,), jnp.int32),
            pltpu.VMEM((_K,), jnp.int32),
            pltpu.VMEM((_K,), jnp.int32),
            pltpu.VMEM((_K, _D), jnp.float32),
            pltpu.VMEM((_K, _D), jnp.float32),
            pltpu.VMEM((_K, _D), jnp.float32),
            pltpu.VMEM((_K, _D), jnp.float32),
            pltpu.VMEM_SHARED((_N_PAD, _D), jnp.float32),
        ] + [pltpu.SemaphoreType.DMA] * 7,
    )
    return f(src, dst, node_feat, w_eff, zeros)


def _combine_body(p0_ref, p1_ref, p2_ref, p3_ref, nf_ref, wsc_ref, o_ref):
    sc = jnp.dot(nf_ref[...], wsc_ref[...], preferred_element_type=jnp.float32)
    s = (p0_ref[...] + p1_ref[...]) + (p2_ref[...] + p3_ref[...])
    o_ref[...] = s + sc * (1.0 / math.sqrt(_D))


def _combine(parts, node_feat, Wsc):
    bn = 2000
    grid = _N_NODES // bn
    blk = pl.BlockSpec((bn, _D), lambda i: (i, 0))
    return pl.pallas_call(
        _combine_body,
        grid=(grid,),
        in_specs=[blk, blk, blk, blk, blk,
                  pl.BlockSpec((_D, _D), lambda i: (0, 0))],
        out_specs=blk,
        out_shape=jax.ShapeDtypeStruct((_N_NODES, _D), jnp.float32),
    )(*parts, node_feat, Wsc)


_E_A = 64 * _BM
_E_B = _N_EDGES - _E_A


def kernel(edge_index, node_feat, edge_feat, edge_embed, dim_size,
           W1, b1, W2, b2, W3, b3, Wsc):
    del dim_size
    src = edge_index[0]
    dst = edge_index[1]
    eet, eft = edge_embed.T, edge_feat.T
    zeros = jnp.zeros((_N_PAD, _D), jnp.float32)
    w_a = _radial_weights(eet, eft, W1, b1, W2, b2, W3, b3, _E_A, 0)
    w_b = _radial_weights(eet, eft, W1, b1, W2, b2, W3, b3, _E_B, _E_A // _BM)
    pa = _sc_scatter(src, dst, node_feat, w_a, zeros,
                     _E_A // _NW, _E_A // _NW // _K, 0)
    zeros_b = zeros + 0.0 * pa[0][0, 0]
    pb = _sc_scatter(src, dst, node_feat, w_b, zeros_b,
                     _E_B // _NW, _E_B // _NW // _K, _E_A)
    return _combine(pa + pb, node_feat, Wsc)
